# baseline (speedup 1.0000x reference)
import functools

import jax
import jax.numpy as jnp
from jax import lax
from jax.experimental import pallas as pl
from jax.experimental.pallas import tpu as pltpu
from jax.experimental.pallas import tpu_sc as plsc

_WIN = 32
_CH = 32


def _sc_body(img_hbm, pos_hbm, w_hbm, out_hbm, pos_v, w_v, band_v,
             buf0, buf1, si0, si1, so0, so1):
    info = plsc.get_sparse_core_info()
    NC, NS, L = info.num_cores, info.num_subcores, info.num_lanes
    wid = lax.axis_index("s") * NC + lax.axis_index("c")
    n_img = img_hbm.shape[0] // (NC * NS)
    H = img_hbm.shape[1]
    base = wid * n_img
    pltpu.sync_copy(pos_hbm.at[pl.ds(base * 2, n_img * 2)], pos_v)
    pltpu.sync_copy(w_hbm, w_v)
    pos_vec = pos_v[...]
    lanes = lax.iota(jnp.int32, L)

    ch_per_img = H // _CH
    n_ch = n_img * ch_per_img

    def src_slice(c):
        return img_hbm.at[base + c // ch_per_img,
                          pl.ds((c % ch_per_img) * _CH, _CH), :]

    def dst_slice(c):
        return out_hbm.at[base + c // ch_per_img,
                          pl.ds((c % ch_per_img) * _CH, _CH), :]

    pltpu.async_copy(src_slice(0), buf0, si0)
    pltpu.async_copy(src_slice(1), buf1, si1)

    def step(t, carry):
        c0 = 2 * t
        c1 = 2 * t + 1
        pltpu.make_async_copy(src_slice(c0), buf0, si0).wait()
        out0 = pltpu.async_copy(buf0, dst_slice(c0), so0)
        pltpu.make_async_copy(src_slice(c1), buf1, si1).wait()
        out1 = pltpu.async_copy(buf1, dst_slice(c1), so1)

        @pl.when(t + 1 < n_ch // 2)
        def _prefetch():
            nc0 = jnp.minimum(c0 + 2, n_ch - 1)
            nc1 = jnp.minimum(c1 + 2, n_ch - 1)
            out0.wait()
            pltpu.async_copy(src_slice(nc0), buf0, si0)
            out1.wait()
            pltpu.async_copy(src_slice(nc1), buf1, si1)

        @pl.when(t + 1 >= n_ch // 2)
        def _drain():
            out0.wait()
            out1.wait()

        return carry

    lax.fori_loop(0, n_ch // 2, step, 0)

    for j in range(n_img):
        k = base + j
        p0 = pos_vec[2 * j]
        p1 = pos_vec[2 * j + 1]
        pltpu.sync_copy(img_hbm.at[k, pl.ds(p0, _WIN), :], band_v)
        for r in range(_WIN):
            row_idx = jnp.full((L,), r, dtype=jnp.int32)
            for h in range(_WIN // L):
                col_idx = p1 + h * L + lanes
                plsc.store_scatter(
                    band_v, [row_idx, col_idx], w_v[r, pl.ds(h * L, L)]
                )
        pltpu.sync_copy(band_v, out_hbm.at[k, pl.ds(p0, _WIN), :])


def kernel(images, position, W):
    B, H, Wimg = images.shape
    info = plsc.get_sparse_core_info()
    n_img = B // (info.num_cores * info.num_subcores)
    mesh = plsc.VectorSubcoreMesh(core_axis_name="c", subcore_axis_name="s")
    f = functools.partial(
        pl.kernel,
        out_type=jax.ShapeDtypeStruct(images.shape, images.dtype),
        mesh=mesh,
        scratch_types=[
            pltpu.VMEM((n_img * 2,), jnp.int32),
            pltpu.VMEM((_WIN, _WIN), jnp.float32),
            pltpu.VMEM((_WIN, Wimg), jnp.float32),
            pltpu.VMEM((_CH, Wimg), jnp.float32),
            pltpu.VMEM((_CH, Wimg), jnp.float32),
            pltpu.SemaphoreType.DMA,
            pltpu.SemaphoreType.DMA,
            pltpu.SemaphoreType.DMA,
            pltpu.SemaphoreType.DMA,
        ],
        compiler_params=pltpu.CompilerParams(
            use_tc_tiling_on_sc=False, needs_layout_passes=False
        ),
    )(_sc_body)
    return f(images, position.astype(jnp.int32).reshape(-1), W)


# speedup vs baseline: 4.0753x; 4.0753x over previous
import jax
import jax.numpy as jnp
from jax.experimental import pallas as pl
from jax.experimental.pallas import tpu as pltpu

_WIN = 32
_SLAB = _WIN + 8


_BI = 8


def _body(pos_ref, img_ref, wpad_ref, out_ref):
    i = pl.program_id(0)
    out_ref[...] = img_ref[...]
    for j in range(_BI):
        p0 = pos_ref[i * _BI + j, 0]
        p1 = pos_ref[i * _BI + j, 1]
        a = pl.multiple_of((p0 // 8) * 8, 8)
        r = p0 - a
        slab = img_ref[j, pl.ds(a, _SLAB), :]
        Wimg = slab.shape[1]
        ri = jax.lax.broadcasted_iota(jnp.int32, (_SLAB, Wimg), 0)
        ci = jax.lax.broadcasted_iota(jnp.int32, (_SLAB, Wimg), 1)
        mask = (ri >= r) & (ri < r + _WIN) & (ci >= p1) & (ci < p1 + _WIN)
        w_shift = pltpu.roll(pltpu.roll(wpad_ref[...], r, 0), p1, 1)
        out_ref[j, pl.ds(a, _SLAB), :] = jnp.where(mask, w_shift, slab)


def kernel(images, position, W):
    B, H, Wimg = images.shape
    wpad = jnp.zeros((_SLAB, Wimg), dtype=W.dtype).at[:_WIN, :_WIN].set(W)
    grid_spec = pltpu.PrefetchScalarGridSpec(
        num_scalar_prefetch=1,
        grid=(B // _BI,),
        in_specs=[
            pl.BlockSpec((_BI, H, Wimg), lambda i, pos: (i, 0, 0)),
            pl.BlockSpec((_SLAB, Wimg), lambda i, pos: (0, 0)),
        ],
        out_specs=pl.BlockSpec((_BI, H, Wimg), lambda i, pos: (i, 0, 0)),
    )
    return pl.pallas_call(
        _body,
        grid_spec=grid_spec,
        out_shape=jax.ShapeDtypeStruct(images.shape, images.dtype),
    )(position.astype(jnp.int32), images, wpad)
